# Initial kernel scaffold; baseline (speedup 1.0000x reference)
import functools

import jax
import jax.numpy as jnp
from jax import lax
from jax.experimental import pallas as pl
from jax.experimental.pallas import tpu as pltpu
from jax.experimental.pallas import tpu_sc as plsc

NUM_EMB = 1000000
DIM = 32

B_TOTAL = 4096 * 200
IDX_MINOR = 128
CHUNK = 1024
STREAMS_PER_CHUNK = CHUNK // IDX_MINOR


def _make_kernel():
    info = plsc.get_sparse_core_info()
    NC, NS = info.num_cores, info.num_subcores
    NW = NC * NS
    rows_per_w = B_TOTAL // NW
    chunks_per_w = rows_per_w // CHUNK
    idx_rows_per_chunk = CHUNK // IDX_MINOR

    mesh = plsc.VectorSubcoreMesh(core_axis_name="c", subcore_axis_name="s")

    @functools.partial(
        pl.kernel,
        mesh=mesh,
        out_type=jax.ShapeDtypeStruct((B_TOTAL, DIM), jnp.float32),
        scratch_types=[
            pltpu.VMEM((idx_rows_per_chunk, IDX_MINOR), jnp.int32),
            pltpu.VMEM((CHUNK, DIM), jnp.float32),
            pltpu.SemaphoreType.DMA,
        ],
    )
    def emb(idx_hbm, table_hbm, out_hbm, idx_v, rows_v, sem):
        wid = lax.axis_index("s") * NC + lax.axis_index("c")
        w_row0 = wid * rows_per_w

        def body(i, carry):
            row0 = w_row0 + i * CHUNK
            pltpu.sync_copy(
                idx_hbm.at[pl.ds(row0 // IDX_MINOR, idx_rows_per_chunk)],
                idx_v,
            )
            copies = []
            for j in range(STREAMS_PER_CHUNK):
                copies.append(
                    pltpu.async_copy(
                        table_hbm.at[idx_v.at[j]],
                        rows_v.at[pl.ds(j * IDX_MINOR, IDX_MINOR)],
                        sem,
                    )
                )
            for c in copies:
                c.wait()
            pltpu.sync_copy(rows_v, out_hbm.at[pl.ds(row0, CHUNK)])
            return carry

        lax.fori_loop(0, chunks_per_w, body, 0)

    return emb


_emb_kernel = _make_kernel()


@jax.jit
def kernel(inputs, weight):
    idx2d = inputs.reshape(B_TOTAL // IDX_MINOR, IDX_MINOR).astype(jnp.int32)
    out = _emb_kernel(idx2d, weight)
    return out.reshape(inputs.shape + (DIM,))


# speedup vs baseline: 1.4610x; 1.4610x over previous
import functools

import jax
import jax.numpy as jnp
from jax import lax
from jax.experimental import pallas as pl
from jax.experimental.pallas import tpu as pltpu
from jax.experimental.pallas import tpu_sc as plsc

NUM_EMB = 1000000
DIM = 32

B_TOTAL = 4096 * 200
IDX_MINOR = 128
CHUNK = 1024
STREAMS_PER_CHUNK = CHUNK // IDX_MINOR


def _make_kernel():
    info = plsc.get_sparse_core_info()
    NC, NS = info.num_cores, info.num_subcores
    NW = NC * NS
    rows_per_w = B_TOTAL // NW
    chunks_per_w = rows_per_w // CHUNK
    idx_rows_per_chunk = CHUNK // IDX_MINOR

    mesh = plsc.VectorSubcoreMesh(core_axis_name="c", subcore_axis_name="s")

    @functools.partial(
        pl.kernel,
        mesh=mesh,
        out_type=jax.ShapeDtypeStruct((B_TOTAL, DIM), jnp.float32),
        scratch_types=[
            pltpu.VMEM((idx_rows_per_chunk, IDX_MINOR), jnp.int32),
            pltpu.VMEM((CHUNK, DIM), jnp.float32),
            pltpu.SemaphoreType.DMA,
        ],
        compiler_params=pltpu.CompilerParams(use_tc_tiling_on_sc=False),
    )
    def emb(idx_hbm, table_hbm, out_hbm, idx_v, rows_v, sem):
        wid = lax.axis_index("s") * NC + lax.axis_index("c")
        w_row0 = wid * rows_per_w

        def body(i, carry):
            row0 = pl.multiple_of(w_row0 + i * CHUNK, CHUNK)
            pltpu.sync_copy(
                idx_hbm.at[
                    pl.ds(pl.multiple_of(row0 // IDX_MINOR, 8), idx_rows_per_chunk)
                ],
                idx_v,
            )
            copies = []
            for j in range(STREAMS_PER_CHUNK):
                copies.append(
                    pltpu.async_copy(
                        table_hbm.at[idx_v.at[j]],
                        rows_v.at[pl.ds(j * IDX_MINOR, IDX_MINOR)],
                        sem,
                    )
                )
            for c in copies:
                c.wait()
            pltpu.sync_copy(rows_v, out_hbm.at[pl.ds(row0, CHUNK)])
            return carry

        lax.fori_loop(0, chunks_per_w, body, 0)

    return emb


_emb_kernel = _make_kernel()


@jax.jit
def kernel(inputs, weight):
    idx2d = inputs.reshape(B_TOTAL // IDX_MINOR, IDX_MINOR).astype(jnp.int32)
    out = _emb_kernel(idx2d, weight)
    return out.reshape(inputs.shape + (DIM,))


# speedup vs baseline: 1.4880x; 1.0185x over previous
import functools

import jax
import jax.numpy as jnp
from jax import lax
from jax.experimental import pallas as pl
from jax.experimental.pallas import tpu as pltpu
from jax.experimental.pallas import tpu_sc as plsc

NUM_EMB = 1000000
DIM = 32

B_TOTAL = 4096 * 200
IDX_MINOR = 128
CHUNK = 1280
STREAMS_PER_CHUNK = CHUNK // IDX_MINOR
NBUF = 2


def _make_kernel():
    info = plsc.get_sparse_core_info()
    NC, NS = info.num_cores, info.num_subcores
    NW = NC * NS
    rows_per_w = B_TOTAL // NW
    chunks_per_w = rows_per_w // CHUNK
    idx_rows = CHUNK // IDX_MINOR

    mesh = plsc.VectorSubcoreMesh(core_axis_name="c", subcore_axis_name="s")

    @functools.partial(
        pl.kernel,
        mesh=mesh,
        out_type=jax.ShapeDtypeStruct((B_TOTAL, DIM), jnp.float32),
        scratch_types=[
            pltpu.VMEM((NBUF, idx_rows, IDX_MINOR), jnp.int32),
            pltpu.VMEM((NBUF, CHUNK, DIM), jnp.float32),
            pltpu.SemaphoreType.DMA((NBUF,)),
            pltpu.SemaphoreType.DMA((NBUF,)),
        ],
        compiler_params=pltpu.CompilerParams(use_tc_tiling_on_sc=False),
    )
    def emb(idx_hbm, table_hbm, out_hbm, idx_v, rows_v, gsem, osem):
        wid = lax.axis_index("s") * NC + lax.axis_index("c")
        w_row0 = wid * rows_per_w

        def fire_gather(c, b):
            row0 = pl.multiple_of(w_row0 + c * CHUNK, CHUNK)
            pltpu.sync_copy(
                idx_hbm.at[pl.ds(pl.multiple_of(row0 // IDX_MINOR, 8), idx_rows)],
                idx_v.at[b],
            )
            for j in range(STREAMS_PER_CHUNK):
                pltpu.async_copy(
                    table_hbm.at[idx_v.at[b].at[j]],
                    rows_v.at[b].at[pl.ds(j * IDX_MINOR, IDX_MINOR)],
                    gsem.at[b],
                )

        def wait_gather(b):
            pltpu.make_async_copy(
                rows_v.at[b], out_hbm.at[pl.ds(0, CHUNK)], gsem.at[b]
            ).wait()

        def fire_out(c, b):
            row0 = pl.multiple_of(w_row0 + c * CHUNK, CHUNK)
            pltpu.async_copy(
                rows_v.at[b], out_hbm.at[pl.ds(row0, CHUNK)], osem.at[b]
            )

        def wait_out(b):
            pltpu.make_async_copy(
                rows_v.at[b], out_hbm.at[pl.ds(0, CHUNK)], osem.at[b]
            ).wait()

        fire_gather(0, 0)

        def body(g, carry):
            b = lax.rem(g, NBUF)
            b_next = lax.rem(g + 1, NBUF)

            @pl.when(g >= 1)
            def _():
                wait_out(b_next)

            @pl.when(g + 1 < chunks_per_w)
            def _():
                fire_gather(g + 1, b_next)

            wait_gather(b)
            fire_out(g, b)
            return carry

        lax.fori_loop(0, chunks_per_w, body, 0)
        wait_out(lax.rem(chunks_per_w - 1, NBUF))

    return emb


_emb_kernel = _make_kernel()


@jax.jit
def kernel(inputs, weight):
    idx2d = inputs.reshape(B_TOTAL // IDX_MINOR, IDX_MINOR).astype(jnp.int32)
    out = _emb_kernel(idx2d, weight)
    return out.reshape(inputs.shape + (DIM,))


# speedup vs baseline: 1.4921x; 1.0028x over previous
import functools

import jax
import jax.numpy as jnp
from jax import lax
from jax.experimental.layout import Format, Layout
from jax.experimental import pallas as pl
from jax.experimental.pallas import tpu as pltpu
from jax.experimental.pallas import tpu_sc as plsc

NUM_EMB = 1000000
DIM = 32

BATCH = 4096
SEQ = 200
NB = 8
NBUF = 2


def _make_kernel():
    info = plsc.get_sparse_core_info()
    NC, NS = info.num_cores, info.num_subcores
    NW = NC * NS
    batches_per_w = BATCH // NW
    chunks_per_w = batches_per_w // NB

    mesh = plsc.VectorSubcoreMesh(core_axis_name="c", subcore_axis_name="s")

    @functools.partial(
        pl.kernel,
        mesh=mesh,
        out_type=jax.ShapeDtypeStruct((BATCH, SEQ, DIM), jnp.float32),
        scratch_types=[
            pltpu.VMEM((NBUF, NB, SEQ), jnp.int32),
            pltpu.VMEM((NBUF, NB, SEQ, DIM), jnp.float32),
            pltpu.SemaphoreType.DMA((NBUF,)),
            pltpu.SemaphoreType.DMA((NBUF,)),
        ],
        compiler_params=pltpu.CompilerParams(use_tc_tiling_on_sc=False),
    )
    def emb(idx_hbm, table_hbm, out_hbm, idx_v, rows_v, gsem, osem):
        wid = lax.axis_index("s") * NC + lax.axis_index("c")
        w_batch0 = wid * batches_per_w

        def fire_gather(c, b):
            batch0 = pl.multiple_of(w_batch0 + c * NB, NB)
            pltpu.sync_copy(idx_hbm.at[pl.ds(batch0, NB)], idx_v.at[b])
            for j in range(NB):
                pltpu.async_copy(
                    table_hbm.at[idx_v.at[b].at[j]],
                    rows_v.at[b].at[j],
                    gsem.at[b],
                )

        def wait_gather(b):
            pltpu.make_async_copy(
                rows_v.at[b], out_hbm.at[pl.ds(0, NB)], gsem.at[b]
            ).wait()

        def fire_out(c, b):
            batch0 = pl.multiple_of(w_batch0 + c * NB, NB)
            pltpu.async_copy(
                rows_v.at[b], out_hbm.at[pl.ds(batch0, NB)], osem.at[b]
            )

        def wait_out(b):
            pltpu.make_async_copy(
                rows_v.at[b], out_hbm.at[pl.ds(0, NB)], osem.at[b]
            ).wait()

        fire_gather(0, 0)

        def body(g, carry):
            b = lax.rem(g, NBUF)
            b_next = lax.rem(g + 1, NBUF)

            @pl.when(g >= 1)
            def _():
                wait_out(b_next)

            @pl.when(g + 1 < chunks_per_w)
            def _():
                fire_gather(g + 1, b_next)

            wait_gather(b)
            fire_out(g, b)
            return carry

        lax.fori_loop(0, chunks_per_w, body, 0)
        wait_out(lax.rem(chunks_per_w - 1, NBUF))

    return emb


_emb_kernel = _make_kernel()


@jax.jit
def kernel(inputs, weight):
    return _emb_kernel(inputs.astype(jnp.int32), weight)
